# baseline (speedup 1.0000x reference)
import functools

import jax
import jax.numpy as jnp
from jax import lax
from jax.experimental import pallas as pl
from jax.experimental.pallas import tpu as pltpu
from jax.experimental.pallas import tpu_sc as plsc

B = 16
S = 512
D = 768
L = 16
NV = D // L
EPS = 1e-12

_info = plsc.get_sparse_core_info()
NC = _info.num_cores
NS = _info.num_subcores
NW = NC * NS

S_TC = S // 2
S_PER_W = (S // 2) // NW
CH = 32
NCH_RAW = B * S_PER_W // CH
NCH = 2 * NCH_RAW
NBUF = 4
SL_PER_CH = CH // B
NACC = 8
JB = 8


_GATHER_DNUMS = lax.GatherDimensionNumbers(
    offset_dims=(), collapsed_slice_dims=(0,), start_index_map=(0,))


def _lane_shuffle(v, perm):
    return lax.gather(v, perm.reshape(L, 1), _GATHER_DNUMS, slice_sizes=(1,),
                      mode=lax.GatherScatterMode.PROMISE_IN_BOUNDS)


def _allreduce_sum(v):
    for k in (8, 4, 2, 1):
        perm = lax.iota(jnp.int32, L) ^ k
        v = v + _lane_shuffle(v, perm)
    return v


def _rsqrt_vec(x):
    i = plsc.bitcast(x, jnp.int32)
    i = jnp.int32(0x5F3759DF) - lax.shift_right_logical(i, 1)
    y = plsc.bitcast(i, jnp.float32)
    for _ in range(3):
        y = y * (jnp.float32(1.5) - jnp.float32(0.5) * x * y * y)
    return y


@functools.partial(
    pl.kernel,
    out_type=[
        jax.ShapeDtypeStruct((B * S, D), jnp.float32),
        jax.ShapeDtypeStruct((B * S_TC, D), jnp.float32),
    ],
    mesh=plsc.VectorSubcoreMesh(core_axis_name="c", subcore_axis_name="s"),
    compiler_params=pltpu.CompilerParams(needs_layout_passes=False),
    scratch_types=(
        [
            pltpu.VMEM((NCH, CH), jnp.int32),
            pltpu.VMEM((S_PER_W, D), jnp.float32),
            pltpu.VMEM((NCH_RAW, CH), jnp.int32),
            pltpu.VMEM((NCH_RAW, CH), jnp.int32),
            pltpu.VMEM((D,), jnp.float32),
            pltpu.VMEM((D,), jnp.float32),
            pltpu.VMEM((CH, L), jnp.float32),
            pltpu.VMEM((CH, L), jnp.float32),
            pltpu.VMEM((NBUF * CH, D), jnp.float32),
            pltpu.SemaphoreType.DMA,
            pltpu.SemaphoreType.DMA,
            pltpu.SemaphoreType.DMA,
        ]
    ),
)
def _embed_hybrid(ids_hbm, tok_hbm, pos_hbm, gam_hbm, bet_hbm,
                  out_hbm, raw_hbm,
                  idx_v, pos_v, dst_f, dst_r, g_v, b_v, rstd_v, nm_v,
                  buf, gsem, rsem, fsem):
    w = lax.axis_index("s") * NC + lax.axis_index("c")

    pltpu.sync_copy(ids_hbm.at[w], idx_v)
    pltpu.sync_copy(pos_hbm.at[pl.ds(S_TC + w * S_PER_W, S_PER_W)], pos_v)
    pltpu.sync_copy(gam_hbm, g_v)
    pltpu.sync_copy(bet_hbm, b_v)

    lane = lax.iota(jnp.int32, L)
    for c in range(NCH_RAW):
        for g in range(SL_PER_CH):
            s_raw = w * S_PER_W + c * SL_PER_CH + g
            dst_r[c, pl.ds(g * L, L)] = lane * S_TC + s_raw
            dst_f[c, pl.ds(g * L, L)] = lane * S + (S_TC + s_raw)

    def _tok_copy(c):
        base = (c % NBUF) * CH
        return pltpu.make_async_copy(
            tok_hbm.at[idx_v.at[c]], buf.at[pl.ds(base, CH)], gsem)

    def _raw_scatter(c):
        base = (c % NBUF) * CH
        return pltpu.make_async_copy(
            buf.at[pl.ds(base, CH)], raw_hbm.at[dst_r.at[c]], rsem)

    def _fused_scatter(c):
        base = (c % NBUF) * CH
        return pltpu.make_async_copy(
            buf.at[pl.ds(base, CH)], out_hbm.at[dst_f.at[c - NCH_RAW]], fsem)

    def _tree_sum(vs):
        while len(vs) > 1:
            vs = [a + b for a, b in zip(vs[::2], vs[1::2])]
        return vs[0]

    def compute_posadd(cf, base):
        @plsc.parallel_loop(0, CH, unroll=2)
        def pa_row(r0):
            r = base + r0
            p = cf * SL_PER_CH + r0 // B
            for j in range(NV):
                plsc.addupdate(
                    buf.at[r, pl.ds(j * L, L)], pos_v[p, pl.ds(j * L, L)])

    def compute_stats(base):
        @plsc.parallel_loop(0, CH)
        def stat_row(r0):
            r = base + r0
            accs = [jnp.zeros((L,), jnp.float32) for _ in range(NACC)]
            accq = [jnp.zeros((L,), jnp.float32) for _ in range(NACC)]
            for j in range(NV):
                x = buf[r, pl.ds(j * L, L)]
                accs[j % NACC] = accs[j % NACC] + x
                accq[j % NACC] = accq[j % NACC] + x * x
            mean_v = _allreduce_sum(_tree_sum(accs)) * jnp.float32(1.0 / D)
            msq_v = _allreduce_sum(_tree_sum(accq)) * jnp.float32(1.0 / D)
            var_v = jnp.maximum(msq_v - mean_v * mean_v, jnp.float32(0.0))
            rstd = _rsqrt_vec(var_v + jnp.float32(EPS))
            rstd_v[r0, pl.ds(0, L)] = rstd
            nm_v[r0, pl.ds(0, L)] = -mean_v * rstd

    def compute_apply(base):
        for jb in range(0, NV, JB):
            gs = [g_v[pl.ds((jb + t) * L, L)] for t in range(JB)]
            bs = [b_v[pl.ds((jb + t) * L, L)] for t in range(JB)]

            @plsc.parallel_loop(0, CH, unroll=2)
            def apply_rows(r0, jb=jb, gs=gs, bs=bs):
                r = base + r0
                rstd = rstd_v[r0, pl.ds(0, L)]
                nm = nm_v[r0, pl.ds(0, L)]
                for t in range(JB):
                    x = buf[r, pl.ds((jb + t) * L, L)]
                    y = x * rstd + nm
                    buf[r, pl.ds((jb + t) * L, L)] = y * gs[t] + bs[t]

    _tok_copy(0).start()
    _tok_copy(1).start()

    def chunk_body(c, _):
        base = (c % NBUF) * CH
        _tok_copy(c).wait()

        @pl.when(c >= NCH_RAW)
        def _():
            compute_posadd(c - NCH_RAW, base)
            compute_stats(base)

        @pl.when(c + 2 < NCH)
        def _():
            @pl.when(c - 2 >= 0)
            def _():
                _raw_scatter(c - 2).wait()

            _tok_copy(c + 2).start()

        @pl.when(c >= NCH_RAW)
        def _():
            compute_apply(base)
            _fused_scatter(c).start()

        @pl.when(c < NCH_RAW)
        def _():
            _raw_scatter(c).start()

        return 0

    lax.fori_loop(0, NCH, chunk_body, 0)
    for c in range(NCH_RAW, NCH):
        _fused_scatter(c).wait()


def _ln_fin_body(prev_ref, x_ref, pos_ref, g_ref, b_ref, o_ref):
    del prev_ref
    e = x_ref[...] + pos_ref[...]
    mean = jnp.mean(e, axis=1, keepdims=True)
    var = jnp.mean(jnp.square(e - mean), axis=1, keepdims=True)
    o_ref[...] = (e - mean) * lax.rsqrt(var + EPS) * g_ref[...] + b_ref[...]


_ln_fin = pl.pallas_call(
    _ln_fin_body,
    grid=(B,),
    in_specs=[
        pl.BlockSpec(memory_space=pl.ANY),
        pl.BlockSpec((S_TC, D), lambda b: (b, 0)),
        pl.BlockSpec((S_TC, D), lambda b: (0, 0)),
        pl.BlockSpec((1, D), lambda b: (0, 0)),
        pl.BlockSpec((1, D), lambda b: (0, 0)),
    ],
    out_specs=pl.BlockSpec((S_TC, D), lambda b: (2 * b, 0)),
    out_shape=jax.ShapeDtypeStruct((B * S, D), jnp.float32),
    input_output_aliases={0: 0},
)


def kernel(input_ids, token_table, pos_table, ln_gamma, ln_beta):
    ids_t = jnp.transpose(input_ids)
    ids_raw = ids_t[:S_TC].reshape(NW, NCH_RAW, CH)
    ids_fus = ids_t[S_TC:].reshape(NW, NCH_RAW, CH)
    ids_all = jnp.concatenate([ids_raw, ids_fus], axis=1)
    out0, emb_tc = _embed_hybrid(
        ids_all, token_table, pos_table, ln_gamma, ln_beta)
    out = _ln_fin(out0, emb_tc, pos_table,
                  ln_gamma.reshape(1, D), ln_beta.reshape(1, D))
    return out.reshape(B, S, D)


# speedup vs baseline: 1.2544x; 1.2544x over previous
import functools

import jax
import jax.numpy as jnp
from jax import lax
from jax.experimental import pallas as pl
from jax.experimental.pallas import tpu as pltpu
from jax.experimental.pallas import tpu_sc as plsc

B = 16
S = 512
D = 768
BS = B * S
EPS = 1e-12

_info = plsc.get_sparse_core_info()
NC = _info.num_cores
NS = _info.num_subcores
NW = NC * NS

K = 1
SLICE = BS // K
TOK_PER_W = SLICE // NW
CH = 32
NCH = TOK_PER_W // CH
NBUF = 4

TBLK = 512
NBLK = SLICE // TBLK


@functools.partial(
    pl.kernel,
    out_type=jax.ShapeDtypeStruct((SLICE, D), jnp.float32),
    mesh=plsc.VectorSubcoreMesh(core_axis_name="c", subcore_axis_name="s"),
    compiler_params=pltpu.CompilerParams(needs_layout_passes=False),
    scratch_types=(
        [pltpu.VMEM((NCH, CH), jnp.int32)]
        + [pltpu.VMEM((CH, D), jnp.float32) for _ in range(NBUF)]
        + [pltpu.SemaphoreType.DMA for _ in range(2 * NBUF)]
    ),
)
def _gather_sc(ids_hbm, tok_hbm, out_hbm, idx_v, *rest):
    bufs = list(rest[:NBUF])
    gsem = list(rest[NBUF:2 * NBUF])
    ssem = list(rest[2 * NBUF:])

    w = lax.axis_index("s") * NC + lax.axis_index("c")
    base = w * TOK_PER_W

    pltpu.sync_copy(ids_hbm.at[w], idx_v)

    def start_gather(c):
        return pltpu.async_copy(
            tok_hbm.at[idx_v.at[c]], bufs[c % NBUF], gsem[c % NBUF])

    def start_out(c):
        return pltpu.async_copy(
            bufs[c % NBUF], out_hbm.at[pl.ds(base + c * CH, CH)],
            ssem[c % NBUF])

    ghandles = {}
    shandles = {}
    for c in range(min(NBUF, NCH)):
        ghandles[c] = start_gather(c)
    for c in range(NCH):
        ghandles[c].wait()
        shandles[c] = start_out(c)
        n = c + NBUF
        if n < NCH:
            shandles[n - NBUF].wait()
            ghandles[n] = start_gather(n)
    for c in range(max(0, NCH - NBUF), NCH):
        shandles[c].wait()


def _ln_math(x, pos, g, b):
    e = x + pos
    mean = jnp.mean(e, axis=1, keepdims=True)
    var = jnp.mean(jnp.square(e - mean), axis=1, keepdims=True)
    return (e - mean) * lax.rsqrt(var + EPS) * g + b


def _ln_head_body(x_ref, pos_ref, g_ref, b_ref, o_ref):
    o_ref[...] = _ln_math(x_ref[...], pos_ref[...], g_ref[...], b_ref[...])


_DATA_SPECS = [
    pl.BlockSpec((TBLK, D), lambda i: (i, 0)),
    pl.BlockSpec((S, D), lambda i: (0, 0)),
    pl.BlockSpec((1, D), lambda i: (0, 0)),
    pl.BlockSpec((1, D), lambda i: (0, 0)),
]

_ln_head = pl.pallas_call(
    _ln_head_body,
    grid=(NBLK,),
    in_specs=_DATA_SPECS,
    out_specs=pl.BlockSpec((TBLK, D), lambda i: (i, 0)),
    out_shape=jax.ShapeDtypeStruct((BS, D), jnp.float32),
)

def kernel(input_ids, token_table, pos_table, ln_gamma, ln_beta):
    ids_g = input_ids.reshape(K, NW, NCH, CH)
    g2 = ln_gamma.reshape(1, D)
    b2 = ln_beta.reshape(1, D)
    emb = _gather_sc(ids_g[0], token_table)
    out = _ln_head(emb, pos_table, g2, b2)
    return out.reshape(B, S, D)
